# baseline (speedup 1.0000x reference)
import functools

import jax
import jax.numpy as jnp
from jax.experimental import pallas as pl

E = 160000
T = 480000
H = 128
INT = 64

BE = 2000
BT = 8000


def _silu(x):
    return x * jax.nn.sigmoid(x)


def _stage_a_body(x1_ref, rbf0_ref, wji_ref, bji_ref, wkj_ref, bkj_ref,
                  wr1_ref, wr2_ref, wdown_ref, xji_ref, xk_ref):
    x1 = x1_ref[...]
    x_ji = _silu(jnp.dot(x1, wji_ref[...], preferred_element_type=jnp.float32)
                 + bji_ref[...])
    x_kj = _silu(jnp.dot(x1, wkj_ref[...], preferred_element_type=jnp.float32)
                 + bkj_ref[...])
    rbf = jnp.dot(jnp.dot(rbf0_ref[...], wr1_ref[...],
                          preferred_element_type=jnp.float32), wr2_ref[...],
                  preferred_element_type=jnp.float32)
    xji_ref[...] = x_ji
    xk_ref[...] = _silu(jnp.dot(x_kj * rbf, wdown_ref[...],
                                preferred_element_type=jnp.float32))


def _stage_b_body(sbf_ref, ws1_ref, ws2_ref, out_ref):
    out_ref[...] = jnp.dot(jnp.dot(sbf_ref[...], ws1_ref[...],
                                   preferred_element_type=jnp.float32),
                           ws2_ref[...], preferred_element_type=jnp.float32)


def _stage_d_body(seg_ref, xji_ref, x1_ref, rbf0_ref, wup_ref,
                  wb0a_ref, bb0a_ref, wb0b_ref, bb0b_ref,
                  wlin_ref, blin_ref,
                  wa0a_ref, ba0a_ref, wa0b_ref, ba0b_ref,
                  wa1a_ref, ba1a_ref, wa1b_ref, ba1b_ref,
                  wrbf_ref, e1_ref, e2_ref):
    x_kj = _silu(jnp.dot(seg_ref[...], wup_ref[...],
                         preferred_element_type=jnp.float32))
    e1 = xji_ref[...] + x_kj
    h = _silu(jnp.dot(e1, wb0a_ref[...], preferred_element_type=jnp.float32)
              + bb0a_ref[...])
    e1 = e1 + _silu(jnp.dot(h, wb0b_ref[...],
                            preferred_element_type=jnp.float32) + bb0b_ref[...])
    e1 = _silu(jnp.dot(e1, wlin_ref[...], preferred_element_type=jnp.float32)
               + blin_ref[...]) + x1_ref[...]
    h = _silu(jnp.dot(e1, wa0a_ref[...], preferred_element_type=jnp.float32)
              + ba0a_ref[...])
    e1 = e1 + _silu(jnp.dot(h, wa0b_ref[...],
                            preferred_element_type=jnp.float32) + ba0b_ref[...])
    h = _silu(jnp.dot(e1, wa1a_ref[...], preferred_element_type=jnp.float32)
              + ba1a_ref[...])
    e1 = e1 + _silu(jnp.dot(h, wa1b_ref[...],
                            preferred_element_type=jnp.float32) + ba1b_ref[...])
    e1_ref[...] = e1
    e2_ref[...] = jnp.dot(rbf0_ref[...], wrbf_ref[...],
                          preferred_element_type=jnp.float32) * e1


def _full(shape):
    return pl.BlockSpec(shape, lambda i: tuple(0 for _ in shape))


def kernel(x1, x2, rbf0, sbf, t, idx_kj, idx_ji, W_rbf1, W_rbf2, W_sbf1,
           W_sbf2, W_rbf, W_kj, b_kj, W_ji, b_ji, W_down, W_up, Wb0a, bb0a,
           Wb0b, bb0b, W_lin, b_lin, Wa0a, ba0a, Wa0b, ba0b, Wa1a, ba1a,
           Wa1b, ba1b):
    idx_kj = idx_kj.astype(jnp.int32)
    idx_ji = idx_ji.astype(jnp.int32)
    b_kj2 = b_kj.reshape(1, H)
    b_ji2 = b_ji.reshape(1, H)

    x_ji, xk = pl.pallas_call(
        _stage_a_body,
        grid=(E // BE,),
        in_specs=[
            pl.BlockSpec((BE, H), lambda i: (i, 0)),
            pl.BlockSpec((BE, 6), lambda i: (i, 0)),
            _full((H, H)), _full((1, H)), _full((H, H)), _full((1, H)),
            _full((6, 8)), _full((8, H)), _full((H, INT)),
        ],
        out_specs=[
            pl.BlockSpec((BE, H), lambda i: (i, 0)),
            pl.BlockSpec((BE, INT), lambda i: (i, 0)),
        ],
        out_shape=[
            jax.ShapeDtypeStruct((E, H), jnp.float32),
            jax.ShapeDtypeStruct((E, INT), jnp.float32),
        ],
    )(x1, rbf0, W_ji, b_ji2, W_kj, b_kj2, W_rbf1, W_rbf2, W_down)

    sbf_p = pl.pallas_call(
        _stage_b_body,
        grid=(T // BT,),
        in_specs=[
            pl.BlockSpec((BT, 42), lambda i: (i, 0)),
            _full((42, 8)), _full((8, INT)),
        ],
        out_specs=pl.BlockSpec((BT, INT), lambda i: (i, 0)),
        out_shape=jax.ShapeDtypeStruct((T, INT), jnp.float32),
    )(sbf, W_sbf1, W_sbf2)

    y = jnp.take(xk, idx_kj, axis=0) * sbf_p
    seg = jax.ops.segment_sum(y, idx_ji, num_segments=E)

    e1, e2 = pl.pallas_call(
        _stage_d_body,
        grid=(E // BE,),
        in_specs=[
            pl.BlockSpec((BE, INT), lambda i: (i, 0)),
            pl.BlockSpec((BE, H), lambda i: (i, 0)),
            pl.BlockSpec((BE, H), lambda i: (i, 0)),
            pl.BlockSpec((BE, 6), lambda i: (i, 0)),
            _full((INT, H)),
            _full((H, H)), _full((1, H)), _full((H, H)), _full((1, H)),
            _full((H, H)), _full((1, H)),
            _full((H, H)), _full((1, H)), _full((H, H)), _full((1, H)),
            _full((H, H)), _full((1, H)), _full((H, H)), _full((1, H)),
            _full((6, H)),
        ],
        out_specs=[
            pl.BlockSpec((BE, H), lambda i: (i, 0)),
            pl.BlockSpec((BE, H), lambda i: (i, 0)),
        ],
        out_shape=[
            jax.ShapeDtypeStruct((E, H), jnp.float32),
            jax.ShapeDtypeStruct((E, H), jnp.float32),
        ],
    )(seg, x_ji, x1, rbf0, W_up,
      Wb0a, bb0a.reshape(1, H), Wb0b, bb0b.reshape(1, H),
      W_lin, b_lin.reshape(1, H),
      Wa0a, ba0a.reshape(1, H), Wa0b, ba0b.reshape(1, H),
      Wa1a, ba1a.reshape(1, H), Wa1b, ba1b.reshape(1, H),
      W_rbf)
    return (e1, e2)


# speedup vs baseline: 1.6693x; 1.6693x over previous
import functools

import jax
import jax.numpy as jnp
from jax import lax
from jax.experimental import pallas as pl
from jax.experimental.pallas import tpu as pltpu
from jax.experimental.pallas import tpu_sc as plsc

E = 160000
T = 480000
H = 128
INT = 64

BE = 2000
BT = 8000

NC = 2
NS = 16
NW = NC * NS
TPW = T // NW
G = 120
NCHUNK = TPW // G


def _sc_gather_mul(xk_hbm, sbfp_hbm, idx_hbm, out_hbm,
                   idx_v, rows_v, srows_v, sem, sem2):
    wid = lax.axis_index("s") * NC + lax.axis_index("c")
    base = wid * TPW

    def chunk(g, carry):
        off = base + g * G
        pltpu.sync_copy(idx_hbm.at[pl.ds(off, G)], idx_v)
        cp1 = pltpu.async_copy(xk_hbm.at[idx_v], rows_v, sem)
        cp2 = pltpu.async_copy(sbfp_hbm.at[pl.ds(off, G)], srows_v, sem2)
        cp1.wait()
        cp2.wait()

        def mul_body(r, c):
            for j in range(INT // 16):
                s = pl.ds(j * 16, 16)
                rows_v[r, s] = rows_v[r, s] * srows_v[r, s]
            return c
        lax.fori_loop(0, G, mul_body, 0)
        pltpu.sync_copy(rows_v, out_hbm.at[pl.ds(off, G)])
        return carry

    lax.fori_loop(0, NCHUNK, chunk, 0)


def _sc_sparse_stage(xk, sbf_p, idx_kj):
    mesh = plsc.VectorSubcoreMesh(core_axis_name="c", subcore_axis_name="s")
    f = pl.kernel(
        _sc_gather_mul,
        mesh=mesh,
        compiler_params=pltpu.CompilerParams(use_tc_tiling_on_sc=False),
        out_type=jax.ShapeDtypeStruct((T, INT), jnp.float32),
        scratch_types=[
            pltpu.VMEM((G,), jnp.int32),
            pltpu.VMEM((G, INT), jnp.float32),
            pltpu.VMEM((G, INT), jnp.float32),
            pltpu.SemaphoreType.DMA,
            pltpu.SemaphoreType.DMA,
        ],
    )
    return f(xk, sbf_p, idx_kj)


def _silu(x):
    return x * jax.nn.sigmoid(x)


def _stage_a_body(x1_ref, rbf0_ref, wji_ref, bji_ref, wkj_ref, bkj_ref,
                  wr1_ref, wr2_ref, wdown_ref, xji_ref, xk_ref):
    x1 = x1_ref[...]
    x_ji = _silu(jnp.dot(x1, wji_ref[...], preferred_element_type=jnp.float32)
                 + bji_ref[...])
    x_kj = _silu(jnp.dot(x1, wkj_ref[...], preferred_element_type=jnp.float32)
                 + bkj_ref[...])
    rbf = jnp.dot(jnp.dot(rbf0_ref[...], wr1_ref[...],
                          preferred_element_type=jnp.float32), wr2_ref[...],
                  preferred_element_type=jnp.float32)
    xji_ref[...] = x_ji
    xk_ref[...] = _silu(jnp.dot(x_kj * rbf, wdown_ref[...],
                                preferred_element_type=jnp.float32))


def _stage_b_body(sbf_ref, ws1_ref, ws2_ref, out_ref):
    out_ref[...] = jnp.dot(jnp.dot(sbf_ref[...], ws1_ref[...],
                                   preferred_element_type=jnp.float32),
                           ws2_ref[...], preferred_element_type=jnp.float32)


def _stage_d_body(seg_ref, xji_ref, x1_ref, rbf0_ref, wup_ref,
                  wb0a_ref, bb0a_ref, wb0b_ref, bb0b_ref,
                  wlin_ref, blin_ref,
                  wa0a_ref, ba0a_ref, wa0b_ref, ba0b_ref,
                  wa1a_ref, ba1a_ref, wa1b_ref, ba1b_ref,
                  wrbf_ref, e1_ref, e2_ref):
    x_kj = _silu(jnp.dot(seg_ref[...], wup_ref[...],
                         preferred_element_type=jnp.float32))
    e1 = xji_ref[...] + x_kj
    h = _silu(jnp.dot(e1, wb0a_ref[...], preferred_element_type=jnp.float32)
              + bb0a_ref[...])
    e1 = e1 + _silu(jnp.dot(h, wb0b_ref[...],
                            preferred_element_type=jnp.float32) + bb0b_ref[...])
    e1 = _silu(jnp.dot(e1, wlin_ref[...], preferred_element_type=jnp.float32)
               + blin_ref[...]) + x1_ref[...]
    h = _silu(jnp.dot(e1, wa0a_ref[...], preferred_element_type=jnp.float32)
              + ba0a_ref[...])
    e1 = e1 + _silu(jnp.dot(h, wa0b_ref[...],
                            preferred_element_type=jnp.float32) + ba0b_ref[...])
    h = _silu(jnp.dot(e1, wa1a_ref[...], preferred_element_type=jnp.float32)
              + ba1a_ref[...])
    e1 = e1 + _silu(jnp.dot(h, wa1b_ref[...],
                            preferred_element_type=jnp.float32) + ba1b_ref[...])
    e1_ref[...] = e1
    e2_ref[...] = jnp.dot(rbf0_ref[...], wrbf_ref[...],
                          preferred_element_type=jnp.float32) * e1


def _full(shape):
    return pl.BlockSpec(shape, lambda i: tuple(0 for _ in shape))


def kernel(x1, x2, rbf0, sbf, t, idx_kj, idx_ji, W_rbf1, W_rbf2, W_sbf1,
           W_sbf2, W_rbf, W_kj, b_kj, W_ji, b_ji, W_down, W_up, Wb0a, bb0a,
           Wb0b, bb0b, W_lin, b_lin, Wa0a, ba0a, Wa0b, ba0b, Wa1a, ba1a,
           Wa1b, ba1b):
    idx_kj = idx_kj.astype(jnp.int32)
    idx_ji = idx_ji.astype(jnp.int32)
    b_kj2 = b_kj.reshape(1, H)
    b_ji2 = b_ji.reshape(1, H)

    x_ji, xk = pl.pallas_call(
        _stage_a_body,
        grid=(E // BE,),
        in_specs=[
            pl.BlockSpec((BE, H), lambda i: (i, 0)),
            pl.BlockSpec((BE, 6), lambda i: (i, 0)),
            _full((H, H)), _full((1, H)), _full((H, H)), _full((1, H)),
            _full((6, 8)), _full((8, H)), _full((H, INT)),
        ],
        out_specs=[
            pl.BlockSpec((BE, H), lambda i: (i, 0)),
            pl.BlockSpec((BE, INT), lambda i: (i, 0)),
        ],
        out_shape=[
            jax.ShapeDtypeStruct((E, H), jnp.float32),
            jax.ShapeDtypeStruct((E, INT), jnp.float32),
        ],
    )(x1, rbf0, W_ji, b_ji2, W_kj, b_kj2, W_rbf1, W_rbf2, W_down)

    sbf_p = pl.pallas_call(
        _stage_b_body,
        grid=(T // BT,),
        in_specs=[
            pl.BlockSpec((BT, 42), lambda i: (i, 0)),
            _full((42, 8)), _full((8, INT)),
        ],
        out_specs=pl.BlockSpec((BT, INT), lambda i: (i, 0)),
        out_shape=jax.ShapeDtypeStruct((T, INT), jnp.float32),
    )(sbf, W_sbf1, W_sbf2)

    y = _sc_sparse_stage(xk, sbf_p, idx_kj)
    seg = jax.ops.segment_sum(y, idx_ji, num_segments=E)

    e1, e2 = pl.pallas_call(
        _stage_d_body,
        grid=(E // BE,),
        in_specs=[
            pl.BlockSpec((BE, INT), lambda i: (i, 0)),
            pl.BlockSpec((BE, H), lambda i: (i, 0)),
            pl.BlockSpec((BE, H), lambda i: (i, 0)),
            pl.BlockSpec((BE, 6), lambda i: (i, 0)),
            _full((INT, H)),
            _full((H, H)), _full((1, H)), _full((H, H)), _full((1, H)),
            _full((H, H)), _full((1, H)),
            _full((H, H)), _full((1, H)), _full((H, H)), _full((1, H)),
            _full((H, H)), _full((1, H)), _full((H, H)), _full((1, H)),
            _full((6, H)),
        ],
        out_specs=[
            pl.BlockSpec((BE, H), lambda i: (i, 0)),
            pl.BlockSpec((BE, H), lambda i: (i, 0)),
        ],
        out_shape=[
            jax.ShapeDtypeStruct((E, H), jnp.float32),
            jax.ShapeDtypeStruct((E, H), jnp.float32),
        ],
    )(seg, x_ji, x1, rbf0, W_up,
      Wb0a, bb0a.reshape(1, H), Wb0b, bb0b.reshape(1, H),
      W_lin, b_lin.reshape(1, H),
      Wa0a, ba0a.reshape(1, H), Wa0b, ba0b.reshape(1, H),
      Wa1a, ba1a.reshape(1, H), Wa1b, ba1b.reshape(1, H),
      W_rbf)
    return (e1, e2)
